# Initial kernel scaffold; baseline (speedup 1.0000x reference)
import jax
import jax.numpy as jnp
from jax.experimental import pallas as pl


def kernel(x, edge_index, edge_weight, W1, b1, W2, b2, L1W, L1b, L2W, L2b):
    raise NotImplementedError("write your pallas kernel here")


# speedup vs baseline: 4.1512x; 4.1512x over previous
import functools

import jax
import jax.numpy as jnp
from jax import lax
from jax.experimental import pallas as pl
from jax.experimental.pallas import tpu as pltpu
from jax.experimental.pallas import tpu_sc as plsc

N = 10000
E = 320000
F = 128
NCLASS = 64

NC = 2
NS = 16
NW = NC * NS
EPW = E // NW
CHUNK = 80
NCHUNK = EPW // CHUNK
ZB = 400
NZB = N // ZB


def _spmm_body(sup_hbm, src_hbm, dst_hbm, ew_hbm, zeros_hbm, out_hbm,
               src_v, dst_v, ew_v, rows_v, acc_sh, sem):
    cid = lax.axis_index("c")
    sid = lax.axis_index("s")
    wid = cid * NS + sid

    for k in range((NZB + NS - 1) // NS):
        blk = k * NS + sid
        @pl.when(blk < NZB)
        def _():
            r0 = pl.multiple_of(blk * ZB, 8)
            pltpu.sync_copy(zeros_hbm.at[pl.ds(r0, ZB)],
                            acc_sh.at[pl.ds(r0, ZB)])
    plsc.subcore_barrier()

    def chunk_body(c, carry):
        base = pl.multiple_of(wid * EPW + c * CHUNK, CHUNK)
        pltpu.sync_copy(src_hbm.at[pl.ds(base, CHUNK)], src_v)
        pltpu.sync_copy(dst_hbm.at[pl.ds(base, CHUNK)], dst_v)
        pltpu.sync_copy(ew_hbm.at[pl.ds(base, CHUNK)], ew_v)
        pltpu.async_copy(sup_hbm.at[src_v], rows_v, sem).wait()

        def group_body(g, c2):
            e0 = g * 16
            ew16 = ew_v[pl.ds(e0, 16)]
            for j in range(16):
                wv = jnp.full((16,), ew16[j], jnp.float32)
                for f in range(F // 16):
                    sl = pl.ds(f * 16, 16)
                    rows_v[e0 + j, sl] = rows_v[e0 + j, sl] * wv
            return c2

        lax.fori_loop(0, CHUNK // 16, group_body, 0, unroll=False)
        pltpu.sync_copy(rows_v, acc_sh.at[dst_v], add=True)
        return carry

    lax.fori_loop(0, NCHUNK, chunk_body, 0, unroll=False)
    plsc.subcore_barrier()
    for k in range((NZB + NS - 1) // NS):
        blk = k * NS + sid
        @pl.when(blk < NZB)
        def _():
            r0 = pl.multiple_of(blk * ZB, 8)
            pltpu.sync_copy(acc_sh.at[pl.ds(r0, ZB)],
                            out_hbm.at[cid, pl.ds(r0, ZB)])


_spmm = pl.kernel(
    _spmm_body,
    out_type=jax.ShapeDtypeStruct((NC, N, F), jnp.float32),
    mesh=plsc.VectorSubcoreMesh(core_axis_name="c", subcore_axis_name="s",
                                num_cores=NC, num_subcores=NS),
    scratch_types=[
        pltpu.VMEM((CHUNK,), jnp.int32),
        pltpu.VMEM((CHUNK,), jnp.int32),
        pltpu.VMEM((CHUNK,), jnp.float32),
        pltpu.VMEM((CHUNK, F), jnp.float32),
        pltpu.VMEM_SHARED((N, F), jnp.float32),
        pltpu.SemaphoreType.DMA,
    ],
)


def _mm_kernel(x_ref, w_ref, o_ref):
    o_ref[...] = jnp.dot(x_ref[...], w_ref[...],
                         preferred_element_type=jnp.float32)


def _fuse_kernel(p_ref, b_ref, w_ref, o_ref):
    h = jnp.maximum(p_ref[0] + p_ref[1] + b_ref[...], 0.0)
    o_ref[...] = jnp.dot(h, w_ref[...], preferred_element_type=jnp.float32)


def _heads_kernel(p_ref, b_ref, w_ref, hb_ref, o_ref):
    h = jnp.maximum(p_ref[0] + p_ref[1] + b_ref[...], 0.0)
    o_ref[...] = jnp.dot(h, w_ref[...],
                         preferred_element_type=jnp.float32) + hb_ref[...]


def kernel(x, edge_index, edge_weight, W1, b1, W2, b2, L1W, L1b, L2W, L2b):
    src = edge_index[0]
    dst = edge_index[1]
    zeros = jnp.zeros((N, F), jnp.float32)

    support1 = pl.pallas_call(
        _mm_kernel,
        out_shape=jax.ShapeDtypeStruct((N, F), jnp.float32),
    )(x, W1)

    p = _spmm(support1, src, dst, edge_weight, zeros)

    support2 = pl.pallas_call(
        _fuse_kernel,
        out_shape=jax.ShapeDtypeStruct((N, F), jnp.float32),
    )(p, b1.reshape(1, F), W2)

    q = _spmm(support2, src, dst, edge_weight, zeros)

    Wcat = jnp.concatenate([L1W, L2W], axis=1)
    bcat = jnp.concatenate([L1b, L2b]).reshape(1, 2 * NCLASS)
    out = pl.pallas_call(
        _heads_kernel,
        out_shape=jax.ShapeDtypeStruct((N, 2 * NCLASS), jnp.float32),
    )(q, b2.reshape(1, F), Wcat, bcat)

    return out[:, :NCLASS], out[:, NCLASS:]


# speedup vs baseline: 8.0275x; 1.9338x over previous
import jax
import jax.numpy as jnp
from jax import lax
from jax.experimental import pallas as pl
from jax.experimental.pallas import tpu as pltpu
from jax.experimental.pallas import tpu_sc as plsc

N = 10000
E = 320000
F = 128
NCLASS = 64

NC = 2
NS = 16
NW = NC * NS
EPW = E // NW
CHUNK = 80
NCHUNK = EPW // CHUNK
NPAIR = (NCHUNK - 1) // 2
ZB = 400
NZB = N // ZB


def _spmm_body(sup_hbm, idx_hbm, ew_hbm, zeros_hbm, out_hbm,
               idx_a, idx_b, ew_a, ew_b, rows_a, rows_b, acc_sh,
               isem_a, isem_b, gsem_a, gsem_b, ssem_a, ssem_b):
    cid = lax.axis_index("c")
    sid = lax.axis_index("s")
    wid = cid * NS + sid

    for k in range((NZB + NS - 1) // NS):
        blk = k * NS + sid
        @pl.when(blk < NZB)
        def _():
            r0 = pl.multiple_of(blk * ZB, 8)
            pltpu.sync_copy(zeros_hbm.at[pl.ds(r0, ZB)],
                            acc_sh.at[pl.ds(r0, ZB)])
    plsc.subcore_barrier()

    bufs_a = (idx_a, ew_a, rows_a, isem_a, gsem_a, ssem_a)
    bufs_b = (idx_b, ew_b, rows_b, isem_b, gsem_b, ssem_b)

    def istart(bufs, c):
        idx_v, ew_v, _, isem, _, _ = bufs
        pltpu.async_copy(idx_hbm.at[wid, c], idx_v, isem)
        pltpu.async_copy(ew_hbm.at[wid, c], ew_v, isem)

    def iwait(bufs, c):
        idx_v, ew_v, _, isem, _, _ = bufs
        pltpu.make_async_copy(idx_hbm.at[wid, c], idx_v, isem).wait()
        pltpu.make_async_copy(ew_hbm.at[wid, c], ew_v, isem).wait()

    def gstart(bufs):
        idx_v, _, rows_v, _, gsem, _ = bufs
        pltpu.async_copy(sup_hbm.at[idx_v.at[0]], rows_v, gsem)

    def gwait(bufs):
        idx_v, _, rows_v, _, gsem, _ = bufs
        pltpu.make_async_copy(sup_hbm.at[idx_v.at[0]], rows_v, gsem).wait()

    def sstart(bufs):
        idx_v, _, rows_v, _, _, ssem = bufs
        pltpu.async_copy(rows_v, acc_sh.at[idx_v.at[1]], ssem, add=True)

    def swait(bufs):
        idx_v, _, rows_v, _, _, ssem = bufs
        pltpu.make_async_copy(rows_v, acc_sh.at[idx_v.at[1]], ssem).wait()

    def mult(bufs):
        _, ew_v, rows_v, _, _, _ = bufs

        def group(g, c2):
            ew16 = ew_v[pl.ds(g * 16, 16)]
            for t in range(16):
                wv = jnp.full((16,), ew16[t], jnp.float32)
                e = g * 16 + t
                for f in range(F // 16):
                    sl = pl.ds(f * 16, 16)
                    rows_v[e, sl] = rows_v[e, sl] * wv
            return c2

        lax.fori_loop(0, CHUNK // 16, group, 0, unroll=False)

    istart(bufs_a, 0)
    iwait(bufs_a, 0)
    gstart(bufs_a)
    istart(bufs_b, 1)
    iwait(bufs_b, 1)
    gstart(bufs_b)

    def pair_body(k, carry):
        c0 = 2 * k
        for bufs, c in ((bufs_a, c0), (bufs_b, c0 + 1)):
            gwait(bufs)
            mult(bufs)
            sstart(bufs)

            @pl.when(c + 2 < NCHUNK)
            def _(bufs=bufs, c=c):
                swait(bufs)
                istart(bufs, c + 2)
                iwait(bufs, c + 2)
                gstart(bufs)
        return carry

    lax.fori_loop(0, NPAIR, pair_body, 0, unroll=False)

    gwait(bufs_a)
    mult(bufs_a)
    sstart(bufs_a)
    swait(bufs_b)
    swait(bufs_a)

    plsc.subcore_barrier()
    for k in range((NZB + NS - 1) // NS):
        blk = k * NS + sid
        @pl.when(blk < NZB)
        def _():
            r0 = pl.multiple_of(blk * ZB, 8)
            pltpu.sync_copy(acc_sh.at[pl.ds(r0, ZB)],
                            out_hbm.at[cid, pl.ds(r0, ZB)])


_spmm = pl.kernel(
    _spmm_body,
    out_type=jax.ShapeDtypeStruct((NC, N, F), jnp.float32),
    mesh=plsc.VectorSubcoreMesh(core_axis_name="c", subcore_axis_name="s",
                                num_cores=NC, num_subcores=NS),
    scratch_types=[
        pltpu.VMEM((2, CHUNK), jnp.int32),
        pltpu.VMEM((2, CHUNK), jnp.int32),
        pltpu.VMEM((CHUNK,), jnp.float32),
        pltpu.VMEM((CHUNK,), jnp.float32),
        pltpu.VMEM((CHUNK, F), jnp.float32),
        pltpu.VMEM((CHUNK, F), jnp.float32),
        pltpu.VMEM_SHARED((N, F), jnp.float32),
        pltpu.SemaphoreType.DMA,
        pltpu.SemaphoreType.DMA,
        pltpu.SemaphoreType.DMA,
        pltpu.SemaphoreType.DMA,
        pltpu.SemaphoreType.DMA,
        pltpu.SemaphoreType.DMA,
    ],
)


def _mm_kernel(x_ref, w_ref, o_ref):
    o_ref[...] = jnp.dot(x_ref[...], w_ref[...],
                         preferred_element_type=jnp.float32)


def _fuse_kernel(p_ref, b_ref, w_ref, o_ref):
    h = jnp.maximum(p_ref[0] + p_ref[1] + b_ref[...], 0.0)
    o_ref[...] = jnp.dot(h, w_ref[...], preferred_element_type=jnp.float32)


def _heads_kernel(p_ref, b_ref, w_ref, hb_ref, o_ref):
    h = jnp.maximum(p_ref[0] + p_ref[1] + b_ref[...], 0.0)
    o_ref[...] = jnp.dot(h, w_ref[...],
                         preferred_element_type=jnp.float32) + hb_ref[...]


def kernel(x, edge_index, edge_weight, W1, b1, W2, b2, L1W, L1b, L2W, L2b):
    src = edge_index[0]
    dst = edge_index[1]
    comb = jnp.stack([src.reshape(NW, NCHUNK, CHUNK),
                      dst.reshape(NW, NCHUNK, CHUNK)], axis=2)
    ew3 = edge_weight.reshape(NW, NCHUNK, CHUNK)
    zeros = jnp.zeros((N, F), jnp.float32)

    support1 = pl.pallas_call(
        _mm_kernel,
        out_shape=jax.ShapeDtypeStruct((N, F), jnp.float32),
    )(x, W1)

    p = _spmm(support1, comb, ew3, zeros)

    support2 = pl.pallas_call(
        _fuse_kernel,
        out_shape=jax.ShapeDtypeStruct((N, F), jnp.float32),
    )(p, b1.reshape(1, F), W2)

    q = _spmm(support2, comb, ew3, zeros)

    Wcat = jnp.concatenate([L1W, L2W], axis=1)
    bcat = jnp.concatenate([L1b, L2b]).reshape(1, 2 * NCLASS)
    out = pl.pallas_call(
        _heads_kernel,
        out_shape=jax.ShapeDtypeStruct((N, 2 * NCLASS), jnp.float32),
    )(q, b2.reshape(1, F), Wcat, bcat)

    return out[:, :NCLASS], out[:, NCLASS:]


# speedup vs baseline: 11.0778x; 1.3800x over previous
import jax
import jax.numpy as jnp
from jax import lax
from jax.experimental import pallas as pl
from jax.experimental.pallas import tpu as pltpu
from jax.experimental.pallas import tpu_sc as plsc

N = 10000
E = 320000
F = 128
NCLASS = 64

NC = 2
NS = 16
NW = NC * NS
EPW = E // NW
CHUNK = 80
NCHUNK = EPW // CHUNK
NSLOT = 4
NITER = (NCHUNK - 1) // NSLOT
ZB = 400
NZB = N // ZB


def _spmm_body(sup_hbm, src_hbm, dst_hbm, ew_hbm, zeros_hbm, out_hbm,
               *scr):
    srcv = scr[0:4]
    dstv = scr[4:8]
    ewv = scr[8:12]
    rows = scr[12:16]
    acc_sh = scr[16]
    isem = scr[17:21]
    dsem = scr[21:25]
    gsem = scr[25:29]
    ssem = scr[29:33]

    cid = lax.axis_index("c")
    sid = lax.axis_index("s")
    wid = cid * NS + sid

    for k in range((NZB + NS - 1) // NS):
        blk = k * NS + sid
        @pl.when(blk < NZB)
        def _():
            r0 = pl.multiple_of(blk * ZB, 8)
            pltpu.sync_copy(zeros_hbm.at[pl.ds(r0, ZB)],
                            acc_sh.at[pl.ds(r0, ZB)])
    plsc.subcore_barrier()

    def istart(c, s):
        pltpu.async_copy(src_hbm.at[wid, c, 0], srcv[s], isem[s])
        pltpu.async_copy(ew_hbm.at[wid, c, 0], ewv[s], isem[s])

    def iwait(c, s):
        pltpu.make_async_copy(src_hbm.at[wid, c, 0], srcv[s], isem[s]).wait()
        pltpu.make_async_copy(ew_hbm.at[wid, c, 0], ewv[s], isem[s]).wait()

    def dstart(c, s):
        pltpu.async_copy(dst_hbm.at[wid, c, 0], dstv[s], dsem[s])

    def dwait(c, s):
        pltpu.make_async_copy(dst_hbm.at[wid, c, 0], dstv[s], dsem[s]).wait()

    def gstart(s):
        pltpu.async_copy(sup_hbm.at[srcv[s]], rows[s], gsem[s])

    def gwait(s):
        pltpu.make_async_copy(sup_hbm.at[srcv[s]], rows[s], gsem[s]).wait()

    def sstart(s):
        pltpu.async_copy(rows[s], acc_sh.at[dstv[s]], ssem[s], add=True)

    def swait(s):
        pltpu.make_async_copy(rows[s], acc_sh.at[dstv[s]], ssem[s]).wait()

    def mult(s):
        ew_v, rows_v = ewv[s], rows[s]

        def group(g, c2):
            ew16 = ew_v[pl.ds(g * 16, 16)]
            for t in range(16):
                wv = jnp.full((16,), ew16[t], jnp.float32)
                e = g * 16 + t
                for f in range(F // 16):
                    sl = pl.ds(f * 16, 16)
                    rows_v[e, sl] = rows_v[e, sl] * wv
            return c2

        lax.fori_loop(0, CHUNK // 16, group, 0, unroll=False)

    def chunk_step(c, j):
        gwait(j)
        mult(j)
        dwait(c, j)
        sstart(j)

        @pl.when(c + 2 < NCHUNK)
        def _():
            s2 = (j + 2) % NSLOT
            @pl.when(c >= 2)
            def _():
                swait(s2)
            iwait(c + 2, s2)
            dstart(c + 2, s2)
            gstart(s2)

        @pl.when(c + 4 < NCHUNK)
        def _():
            istart(c + 4, j)

    for c0 in range(NSLOT):
        istart(c0, c0)
    dstart(0, 0)
    dstart(1, 1)
    iwait(0, 0)
    gstart(0)
    iwait(1, 1)
    gstart(1)

    def ring_body(k, carry):
        c = k * NSLOT
        for j in range(NSLOT):
            chunk_step(c + j, j)
        return carry

    lax.fori_loop(0, NITER, ring_body, 0, unroll=False)

    chunk_step(NCHUNK - 1, 0)
    swait(1)
    swait(2)
    swait(3)
    swait(0)

    plsc.subcore_barrier()
    for k in range((NZB + NS - 1) // NS):
        blk = k * NS + sid
        @pl.when(blk < NZB)
        def _():
            r0 = pl.multiple_of(blk * ZB, 8)
            pltpu.sync_copy(acc_sh.at[pl.ds(r0, ZB)],
                            out_hbm.at[cid, pl.ds(r0, ZB)])


_spmm = pl.kernel(
    _spmm_body,
    out_type=jax.ShapeDtypeStruct((NC, N, F), jnp.float32),
    mesh=plsc.VectorSubcoreMesh(core_axis_name="c", subcore_axis_name="s",
                                num_cores=NC, num_subcores=NS),
    scratch_types=(
        [pltpu.VMEM((CHUNK,), jnp.int32) for _ in range(4)]
        + [pltpu.VMEM((CHUNK,), jnp.int32) for _ in range(4)]
        + [pltpu.VMEM((CHUNK,), jnp.float32) for _ in range(4)]
        + [pltpu.VMEM((CHUNK, F), jnp.float32) for _ in range(4)]
        + [pltpu.VMEM_SHARED((N, F), jnp.float32)]
        + [pltpu.SemaphoreType.DMA for _ in range(16)]
    ),
)


def _mm_kernel(x_ref, w_ref, o_ref):
    o_ref[...] = jnp.dot(x_ref[...], w_ref[...],
                         preferred_element_type=jnp.float32)


def _fuse_kernel(p_ref, b_ref, w_ref, o_ref):
    h = jnp.maximum(p_ref[0] + p_ref[1] + b_ref[...], 0.0)
    o_ref[...] = jnp.dot(h, w_ref[...], preferred_element_type=jnp.float32)


def _heads_kernel(p_ref, b_ref, w_ref, hb_ref, o_ref):
    h = jnp.maximum(p_ref[0] + p_ref[1] + b_ref[...], 0.0)
    o_ref[...] = jnp.dot(h, w_ref[...],
                         preferred_element_type=jnp.float32) + hb_ref[...]


def kernel(x, edge_index, edge_weight, W1, b1, W2, b2, L1W, L1b, L2W, L2b):
    src3 = edge_index[0].reshape(NW, NCHUNK, 1, CHUNK)
    dst3 = edge_index[1].reshape(NW, NCHUNK, 1, CHUNK)
    ew3 = edge_weight.reshape(NW, NCHUNK, 1, CHUNK)
    zeros = jnp.zeros((N, F), jnp.float32)

    support1 = pl.pallas_call(
        _mm_kernel,
        out_shape=jax.ShapeDtypeStruct((N, F), jnp.float32),
    )(x, W1)

    p = _spmm(support1, src3, dst3, ew3, zeros)

    support2 = pl.pallas_call(
        _fuse_kernel,
        out_shape=jax.ShapeDtypeStruct((N, F), jnp.float32),
    )(p, b1.reshape(1, F), W2)

    q = _spmm(support2, src3, dst3, ew3, zeros)

    Wcat = jnp.concatenate([L1W, L2W], axis=1)
    bcat = jnp.concatenate([L1b, L2b]).reshape(1, 2 * NCLASS)
    out = pl.pallas_call(
        _heads_kernel,
        out_shape=jax.ShapeDtypeStruct((N, 2 * NCLASS), jnp.float32),
    )(q, b2.reshape(1, F), Wcat, bcat)

    return out[:, :NCLASS], out[:, NCLASS:]
